# Initial kernel scaffold; baseline (speedup 1.0000x reference)
import jax
import jax.numpy as jnp
from jax.experimental import pallas as pl


def kernel(z_base_global, domain_ids, feats, W1, b1, W2, b2, alphas):
    raise NotImplementedError("write your pallas kernel here")


# speedup vs baseline: 17.5474x; 17.5474x over previous
import jax
import jax.numpy as jnp
from jax.experimental import pallas as pl
from jax.experimental.pallas import tpu as pltpu

_NUM_DOMAINS = 8
_K_PER = 32
_G = _NUM_DOMAINS * _K_PER
_FEAT_DIM = 128
_HIDDEN = 128
_BLOCK_B = 512


def _fused_body(z_ref, d_ref, f_ref, w1f_ref, w1c_ref, b1_ref, w2_ref,
                b2_ref, al_ref, out_ref):
    z = z_ref[...]
    d = d_ref[...]
    feats = f_ref[...]
    bs = z.shape[0]

    local = jnp.zeros((bs, _K_PER), jnp.float32)
    alpha = jnp.zeros((bs, 1), jnp.float32)
    for c in range(_NUM_DOMAINS):
        sel = d == c
        local = local + jnp.where(sel, z[:, c * _K_PER:(c + 1) * _K_PER], 0.0)
        alpha = alpha + jnp.where(sel, al_ref[0, c], 0.0)

    m = jnp.max(local, axis=1, keepdims=True)
    e = jnp.exp(local - m)
    s = jnp.sum(e, axis=1, keepdims=True)
    p = e / s
    p_max = jnp.max(p, axis=1, keepdims=True)
    entropy = -jnp.sum(p * jnp.log(jnp.clip(p, 1e-12)), axis=1, keepdims=True)
    eq = p == p_max
    iota = jax.lax.broadcasted_iota(jnp.int32, (bs, _K_PER), 1)
    first_idx = jnp.min(jnp.where(eq, iota, _K_PER), axis=1, keepdims=True)
    second = jnp.max(jnp.where(iota == first_idx, -1.0, p), axis=1, keepdims=True)
    margin = p_max - second

    h = jnp.dot(feats, w1f_ref[...], preferred_element_type=jnp.float32)
    h = h + p_max * w1c_ref[0:1, :] + entropy * w1c_ref[1:2, :] + margin * w1c_ref[2:3, :]
    h = jnp.maximum(h + b1_ref[...], 0.0)

    dz = jnp.dot(h, w2_ref[...], preferred_element_type=jnp.float32)
    dz = (dz + b2_ref[0:1, :_K_PER]) * alpha

    for c in range(_NUM_DOMAINS):
        sel = d == c
        out_ref[:, c * _K_PER:(c + 1) * _K_PER] = (
            z[:, c * _K_PER:(c + 1) * _K_PER] + jnp.where(sel, dz, 0.0))


def kernel(z_base_global, domain_ids, feats, W1, b1, W2, b2, alphas):
    B = z_base_global.shape[0]
    d2 = domain_ids.reshape(B, 1)
    w1f = W1[:, :_FEAT_DIM].T
    w1c = jnp.zeros((8, _HIDDEN), jnp.float32).at[:3].set(W1[:, _FEAT_DIM:].T)
    b1r = b1.reshape(1, _HIDDEN)
    w2t = W2.T
    b2r = jnp.zeros((1, 128), jnp.float32).at[0, :_K_PER].set(b2)
    alr = jnp.zeros((1, 128), jnp.float32).at[0, :_NUM_DOMAINS].set(alphas)

    grid = (B // _BLOCK_B,)
    return pl.pallas_call(
        _fused_body,
        grid=grid,
        in_specs=[
            pl.BlockSpec((_BLOCK_B, _G), lambda i: (i, 0)),
            pl.BlockSpec((_BLOCK_B, 1), lambda i: (i, 0)),
            pl.BlockSpec((_BLOCK_B, _FEAT_DIM), lambda i: (i, 0)),
            pl.BlockSpec((_FEAT_DIM, _HIDDEN), lambda i: (0, 0)),
            pl.BlockSpec((8, _HIDDEN), lambda i: (0, 0)),
            pl.BlockSpec((1, _HIDDEN), lambda i: (0, 0)),
            pl.BlockSpec((_HIDDEN, _K_PER), lambda i: (0, 0)),
            pl.BlockSpec((1, 128), lambda i: (0, 0)),
            pl.BlockSpec((1, 128), lambda i: (0, 0)),
        ],
        out_specs=pl.BlockSpec((_BLOCK_B, _G), lambda i: (i, 0)),
        out_shape=jax.ShapeDtypeStruct((B, _G), jnp.float32),
        compiler_params=pltpu.CompilerParams(
            dimension_semantics=("arbitrary",)),
    )(z_base_global, d2, feats, w1f, w1c, b1r, w2t, b2r, alr)
